# baseline (speedup 1.0000x reference)
import functools

import jax
import jax.numpy as jnp
from jax import lax
from jax.experimental import pallas as pl
from jax.experimental.pallas import tpu as pltpu
from jax.experimental.pallas import tpu_sc as plsc

NC = 2
NS = 16
NW = NC * NS

CHUNK = 1024
UNROLL = 8
BATCH = 128
LANES = 16


def _sc_mesh():
  return plsc.VectorSubcoreMesh(
      core_axis_name="c", subcore_axis_name="s", num_cores=NC, num_subcores=NS)


_SC_PARAMS = pltpu.CompilerParams(use_tc_tiling_on_sc=False,
                                 needs_layout_passes=False)


def _make_gather_kernel(n_pad, e_pad):
  t_per_tile = e_pad // NW
  n_chunks = t_per_tile // CHUNK
  rpc = CHUNK // BATCH

  @functools.partial(
      pl.kernel,
      out_type=[jax.ShapeDtypeStruct((e_pad,), jnp.int32),
                jax.ShapeDtypeStruct((NC, n_pad), jnp.float32)],
      mesh=_sc_mesh(),
      compiler_params=_SC_PARAMS,
      scratch_types=[
          pltpu.VMEM((n_pad,), jnp.int32),
          pltpu.VMEM((CHUNK,), jnp.int32),
          pltpu.VMEM((CHUNK,), jnp.int32),
          pltpu.VMEM((CHUNK,), jnp.int32),
          pltpu.VMEM((BATCH,), jnp.float32),
          pltpu.VMEM_SHARED((n_pad,), jnp.float32),
          pltpu.SemaphoreType.DMA,
      ],
  )
  def gather_kernel(table_hbm, src_hbm, dst_hbm, zeros_hbm, d_out, cnt_out,
                    table_v, src_v, dst_v, d_v, ones_v, cnt_sh, sem):
    cid = lax.axis_index("c")
    sid = lax.axis_index("s")
    wid = sid * NC + cid

    @pl.when(sid == 0)
    def _():
      pltpu.sync_copy(zeros_hbm, cnt_sh)

    for i in range(BATCH // LANES):
      ones_v[pl.ds(i * LANES, LANES)] = jnp.ones((LANES,), jnp.float32)

    pltpu.sync_copy(table_hbm, table_v)
    plsc.subcore_barrier()

    def chunk_body(g, carry):
      eb = wid * t_per_tile + g * CHUNK
      pltpu.sync_copy(src_hbm.at[pl.ds(eb, CHUNK)], src_v)
      pltpu.sync_copy(dst_hbm.at[pl.ds(eb, CHUNK)], dst_v)

      handles = [
          pltpu.async_copy(ones_v, cnt_sh.at[dst_v.at[pl.ds(j * BATCH, BATCH)]],
                           sem, add=True)
          for j in range(rpc)
      ]

      def vec_body(k, carry2):
        for u in range(UNROLL):
          o = (k * UNROLL + u) * LANES
          a = plsc.load_gather(table_v, [src_v[pl.ds(o, LANES)]])
          b = plsc.load_gather(table_v, [dst_v[pl.ds(o, LANES)]])
          diff = plsc.bitcast(a, jnp.bfloat16) - plsc.bitcast(b, jnp.bfloat16)
          d_v[pl.ds(o, LANES)] = plsc.bitcast(diff, jnp.int32)
        return carry2

      lax.fori_loop(0, CHUNK // (LANES * UNROLL), vec_body, 0)
      pltpu.sync_copy(d_v, d_out.at[pl.ds(eb, CHUNK)])
      for h in handles:
        h.wait()
      return carry

    lax.fori_loop(0, n_chunks, chunk_body, 0)
    plsc.subcore_barrier()

    @pl.when(sid == 0)
    def _():
      pltpu.sync_copy(cnt_sh, cnt_out.at[cid])

  return gather_kernel


def _make_scatter_kernel(n_pad, e_pad):
  t_per_tile = e_pad // NW
  n_chunks = t_per_tile // CHUNK

  @functools.partial(
      pl.kernel,
      out_type=[jax.ShapeDtypeStruct((2, NC, n_pad), jnp.float32)],
      mesh=_sc_mesh(),
      compiler_params=_SC_PARAMS,
      scratch_types=[
          pltpu.VMEM((CHUNK // BATCH, BATCH), jnp.int32),
          pltpu.VMEM((CHUNK,), jnp.float32),
          pltpu.VMEM((CHUNK,), jnp.float32),
          pltpu.VMEM_SHARED((n_pad,), jnp.float32),
          pltpu.VMEM_SHARED((n_pad,), jnp.float32),
          pltpu.SemaphoreType.DMA,
      ],
  )
  def scatter_kernel(m0_hbm, m1_hbm, dst2d_hbm, zeros_hbm, part_out,
                     dst_v, m0_v, m1_v, acc0_sh, acc1_sh, sem):
    cid = lax.axis_index("c")
    sid = lax.axis_index("s")
    wid = sid * NC + cid
    rpc = CHUNK // BATCH

    @pl.when(sid == 0)
    def _():
      pltpu.sync_copy(zeros_hbm, acc0_sh)

    @pl.when(sid == 1)
    def _():
      pltpu.sync_copy(zeros_hbm, acc1_sh)

    plsc.subcore_barrier()

    def chunk_body(g, carry):
      eb = wid * t_per_tile + g * CHUNK
      pltpu.sync_copy(dst2d_hbm.at[pl.ds(eb // BATCH, rpc)], dst_v)
      pltpu.sync_copy(m0_hbm.at[pl.ds(eb, CHUNK)], m0_v)
      pltpu.sync_copy(m1_hbm.at[pl.ds(eb, CHUNK)], m1_v)
      handles = []
      for j in range(rpc):
        bsl = pl.ds(j * BATCH, BATCH)
        idx = dst_v.at[j]
        handles.append(pltpu.async_copy(m0_v.at[bsl], acc0_sh.at[idx], sem,
                                        add=True))
        handles.append(pltpu.async_copy(m1_v.at[bsl], acc1_sh.at[idx], sem,
                                        add=True))
      for h in handles:
        h.wait()
      return carry

    lax.fori_loop(0, n_chunks, chunk_body, 0)
    plsc.subcore_barrier()

    @pl.when(sid == 0)
    def _():
      pltpu.sync_copy(acc0_sh, part_out.at[0, cid])

    @pl.when(sid == 1)
    def _():
      pltpu.sync_copy(acc1_sh, part_out.at[1, cid])

  return scatter_kernel


def _mlp_block(d_ref, w1t_ref, w2t_ref, w3t_ref, m0_ref, m1_ref):
  w = d_ref[0]
  d0 = lax.bitcast_convert_type(w << 16, jnp.float32)
  d1 = lax.bitcast_convert_type(w & jnp.int32(-65536), jnp.float32)
  h = jnp.tanh(w1t_ref[:, 0:1] * d0 + w1t_ref[:, 1:2] * d1)
  h = jnp.tanh(jnp.dot(w2t_ref[...], h, preferred_element_type=jnp.float32))
  m = jnp.dot(w3t_ref[...], h, preferred_element_type=jnp.float32)
  m0_ref[0] = m[0:1, :]
  m1_ref[0] = m[1:2, :]


def _combine_block(p_ref, c_ref, o_ref):
  cnt = jnp.sum(c_ref[...], axis=(0, 1), keepdims=False)[None, :]
  denom = jnp.maximum(cnt, 1.0)
  o_ref[0:1, :] = jnp.sum(p_ref[:, 0], axis=(0, 1), keepdims=False)[None, :] / denom
  o_ref[1:2, :] = jnp.sum(p_ref[:, 1], axis=(0, 1), keepdims=False)[None, :] / denom


SLABS = 14


def kernel(x, edge_index, batch, t, W1, W2, W3, Wg, E1_w, E1_b, E2_w, E2_b):
  n = x.shape[0]
  e = edge_index.shape[1]

  slab_granule = NW * CHUNK * SLABS
  e_pad = ((e + slab_granule - 1) // slab_granule) * slab_granule
  e_slab = e_pad // SLABS
  n_pad = ((n + 1 + CHUNK - 1) // CHUNK) * CHUNK

  pad = e_pad - e
  src = jnp.concatenate([edge_index[0], jnp.full((pad,), n, jnp.int32)])
  dst = jnp.concatenate([edge_index[1], jnp.full((pad,), n, jnp.int32)])
  xp = jnp.pad(x, ((0, n_pad - n), (0, 0))).astype(jnp.bfloat16)
  b0 = lax.bitcast_convert_type(xp[:, 0], jnp.uint16).astype(jnp.uint32)
  b1 = lax.bitcast_convert_type(xp[:, 1], jnp.uint16).astype(jnp.uint32)
  xpacked = lax.bitcast_convert_type((b1 << 16) | b0, jnp.int32)
  zeros_n = jnp.zeros((n_pad,), jnp.float32)

  gather = _make_gather_kernel(n_pad, e_slab)
  scatter = _make_scatter_kernel(n_pad, e_slab)

  bt = 8192
  nb = e_slab // bt
  mlp = pl.pallas_call(
      _mlp_block,
      grid=(nb,),
      in_specs=[
          pl.BlockSpec((1, 1, bt), lambda i: (i, 0, 0)),
          pl.BlockSpec((64, 2), lambda i: (0, 0)),
          pl.BlockSpec((64, 64), lambda i: (0, 0)),
          pl.BlockSpec((2, 64), lambda i: (0, 0)),
      ],
      out_specs=[
          pl.BlockSpec((1, 1, bt), lambda i: (i, 0, 0)),
          pl.BlockSpec((1, 1, bt), lambda i: (i, 0, 0)),
      ],
      out_shape=[
          jax.ShapeDtypeStruct((nb, 1, bt), jnp.float32),
          jax.ShapeDtypeStruct((nb, 1, bt), jnp.float32),
      ],
  )

  parts = []
  cnts = []
  for sl in range(SLABS):
    src_s = lax.slice(src, (sl * e_slab,), ((sl + 1) * e_slab,))
    dst_s = lax.slice(dst, (sl * e_slab,), ((sl + 1) * e_slab,))
    d, cntp = gather(xpacked, src_s, dst_s, zeros_n)
    m0, m1 = mlp(d.reshape(nb, 1, bt), W1.T, W2.T, W3.T)
    (part,) = scatter(m0.reshape(e_slab), m1.reshape(e_slab),
                      dst_s.reshape(e_slab // BATCH, BATCH), zeros_n)
    parts.append(part)
    cnts.append(cntp)

  part_all = jnp.stack(parts)
  cnt_all = jnp.stack(cnts)

  bn = 2048
  outT = pl.pallas_call(
      _combine_block,
      grid=(n_pad // bn,),
      in_specs=[pl.BlockSpec((SLABS, 2, NC, bn), lambda i: (0, 0, 0, i)),
                pl.BlockSpec((SLABS, NC, bn), lambda i: (0, 0, i))],
      out_specs=pl.BlockSpec((2, bn), lambda i: (0, i)),
      out_shape=jax.ShapeDtypeStruct((2, n_pad), jnp.float32),
  )(part_all, cnt_all)

  return outT.T[:n]


# speedup vs baseline: 1.5041x; 1.5041x over previous
import functools

import jax
import jax.numpy as jnp
from jax import lax
from jax.experimental import pallas as pl
from jax.experimental.pallas import tpu as pltpu
from jax.experimental.pallas import tpu_sc as plsc

NC = 2
NS = 16
NW = NC * NS

CHUNK = 2048
UNROLL = 8
BATCH = 128
LANES = 16


def _sc_mesh():
  return plsc.VectorSubcoreMesh(
      core_axis_name="c", subcore_axis_name="s", num_cores=NC, num_subcores=NS)


_SC_PARAMS = pltpu.CompilerParams(use_tc_tiling_on_sc=False,
                                 needs_layout_passes=False)


def _make_gather_kernel(n_pad, e_pad):
  t_per_tile = e_pad // NW
  n_chunks = t_per_tile // CHUNK
  rpc = CHUNK // BATCH

  @functools.partial(
      pl.kernel,
      out_type=[jax.ShapeDtypeStruct((e_pad,), jnp.int32),
                jax.ShapeDtypeStruct((NC, n_pad), jnp.float32)],
      mesh=_sc_mesh(),
      compiler_params=_SC_PARAMS,
      scratch_types=[
          pltpu.VMEM((n_pad,), jnp.int32),
          pltpu.VMEM((CHUNK,), jnp.int32),
          pltpu.VMEM((CHUNK,), jnp.int32),
          pltpu.VMEM((CHUNK,), jnp.int32),
          pltpu.VMEM((BATCH,), jnp.float32),
          pltpu.VMEM_SHARED((n_pad,), jnp.float32),
          pltpu.SemaphoreType.DMA,
      ],
  )
  def gather_kernel(table_hbm, src_hbm, dst_hbm, zeros_hbm, d_out, cnt_out,
                    table_v, src_v, dst_v, d_v, ones_v, cnt_sh, sem):
    cid = lax.axis_index("c")
    sid = lax.axis_index("s")
    wid = sid * NC + cid

    @pl.when(sid == 0)
    def _():
      pltpu.sync_copy(zeros_hbm, cnt_sh)

    for i in range(BATCH // LANES):
      ones_v[pl.ds(i * LANES, LANES)] = jnp.ones((LANES,), jnp.float32)

    pltpu.sync_copy(table_hbm, table_v)
    plsc.subcore_barrier()

    def chunk_body(g, carry):
      eb = wid * t_per_tile + g * CHUNK
      pltpu.sync_copy(src_hbm.at[pl.ds(eb, CHUNK)], src_v)
      pltpu.sync_copy(dst_hbm.at[pl.ds(eb, CHUNK)], dst_v)

      handles = [
          pltpu.async_copy(ones_v, cnt_sh.at[dst_v.at[pl.ds(j * BATCH, BATCH)]],
                           sem, add=True)
          for j in range(rpc)
      ]

      def vec_body(k, carry2):
        for u in range(UNROLL):
          o = (k * UNROLL + u) * LANES
          a = plsc.load_gather(table_v, [src_v[pl.ds(o, LANES)]])
          b = plsc.load_gather(table_v, [dst_v[pl.ds(o, LANES)]])
          diff = plsc.bitcast(a, jnp.bfloat16) - plsc.bitcast(b, jnp.bfloat16)
          d_v[pl.ds(o, LANES)] = plsc.bitcast(diff, jnp.int32)
        return carry2

      lax.fori_loop(0, CHUNK // (LANES * UNROLL), vec_body, 0)
      pltpu.sync_copy(d_v, d_out.at[pl.ds(eb, CHUNK)])
      for h in handles:
        h.wait()
      return carry

    lax.fori_loop(0, n_chunks, chunk_body, 0)
    plsc.subcore_barrier()

    @pl.when(sid == 0)
    def _():
      pltpu.sync_copy(cnt_sh, cnt_out.at[cid])

  return gather_kernel


def _make_scatter_kernel(n_pad, e_pad):
  t_per_tile = e_pad // NW
  n_chunks = t_per_tile // CHUNK

  @functools.partial(
      pl.kernel,
      out_type=[jax.ShapeDtypeStruct((2, NC, n_pad), jnp.float32)],
      mesh=_sc_mesh(),
      compiler_params=_SC_PARAMS,
      scratch_types=[
          pltpu.VMEM((CHUNK // BATCH, BATCH), jnp.int32),
          pltpu.VMEM((CHUNK,), jnp.float32),
          pltpu.VMEM((CHUNK,), jnp.float32),
          pltpu.VMEM_SHARED((n_pad,), jnp.float32),
          pltpu.VMEM_SHARED((n_pad,), jnp.float32),
          pltpu.SemaphoreType.DMA,
      ],
  )
  def scatter_kernel(m0_hbm, m1_hbm, dst2d_hbm, zeros_hbm, part_out,
                     dst_v, m0_v, m1_v, acc0_sh, acc1_sh, sem):
    cid = lax.axis_index("c")
    sid = lax.axis_index("s")
    wid = sid * NC + cid
    rpc = CHUNK // BATCH

    @pl.when(sid == 0)
    def _():
      pltpu.sync_copy(zeros_hbm, acc0_sh)

    @pl.when(sid == 1)
    def _():
      pltpu.sync_copy(zeros_hbm, acc1_sh)

    plsc.subcore_barrier()

    def chunk_body(g, carry):
      eb = wid * t_per_tile + g * CHUNK
      pltpu.sync_copy(dst2d_hbm.at[pl.ds(eb // BATCH, rpc)], dst_v)
      pltpu.sync_copy(m0_hbm.at[pl.ds(eb, CHUNK)], m0_v)
      pltpu.sync_copy(m1_hbm.at[pl.ds(eb, CHUNK)], m1_v)
      handles = []
      for j in range(rpc):
        bsl = pl.ds(j * BATCH, BATCH)
        idx = dst_v.at[j]
        handles.append(pltpu.async_copy(m0_v.at[bsl], acc0_sh.at[idx], sem,
                                        add=True))
        handles.append(pltpu.async_copy(m1_v.at[bsl], acc1_sh.at[idx], sem,
                                        add=True))
      for h in handles:
        h.wait()
      return carry

    lax.fori_loop(0, n_chunks, chunk_body, 0)
    plsc.subcore_barrier()

    @pl.when(sid == 0)
    def _():
      pltpu.sync_copy(acc0_sh, part_out.at[0, cid])

    @pl.when(sid == 1)
    def _():
      pltpu.sync_copy(acc1_sh, part_out.at[1, cid])

  return scatter_kernel


def _mlp_block(d_ref, w1t_ref, w2t_ref, w3t_ref, m0_ref, m1_ref):
  w = d_ref[0]
  d0 = lax.bitcast_convert_type(w << 16, jnp.float32)
  d1 = lax.bitcast_convert_type(w & jnp.int32(-65536), jnp.float32)
  h = jnp.tanh(w1t_ref[:, 0:1] * d0 + w1t_ref[:, 1:2] * d1)
  h = jnp.tanh(jnp.dot(w2t_ref[...], h, preferred_element_type=jnp.float32))
  m = jnp.dot(w3t_ref[...], h, preferred_element_type=jnp.float32)
  m0_ref[0] = m[0:1, :]
  m1_ref[0] = m[1:2, :]


def _combine_block(p_ref, c_ref, o_ref):
  cnt = jnp.sum(c_ref[...], axis=(0, 1), keepdims=False)[None, :]
  denom = jnp.maximum(cnt, 1.0)
  o_ref[0:1, :] = jnp.sum(p_ref[:, 0], axis=(0, 1), keepdims=False)[None, :] / denom
  o_ref[1:2, :] = jnp.sum(p_ref[:, 1], axis=(0, 1), keepdims=False)[None, :] / denom


SLABS = 7


def kernel(x, edge_index, batch, t, W1, W2, W3, Wg, E1_w, E1_b, E2_w, E2_b):
  n = x.shape[0]
  e = edge_index.shape[1]

  slab_granule = NW * CHUNK * SLABS
  e_pad = ((e + slab_granule - 1) // slab_granule) * slab_granule
  e_slab = e_pad // SLABS
  n_pad = ((n + 1 + CHUNK - 1) // CHUNK) * CHUNK

  pad = e_pad - e
  src = jnp.concatenate([edge_index[0], jnp.full((pad,), n, jnp.int32)])
  dst = jnp.concatenate([edge_index[1], jnp.full((pad,), n, jnp.int32)])
  xp = jnp.pad(x, ((0, n_pad - n), (0, 0))).astype(jnp.bfloat16)
  b0 = lax.bitcast_convert_type(xp[:, 0], jnp.uint16).astype(jnp.uint32)
  b1 = lax.bitcast_convert_type(xp[:, 1], jnp.uint16).astype(jnp.uint32)
  xpacked = lax.bitcast_convert_type((b1 << 16) | b0, jnp.int32)
  zeros_n = jnp.zeros((n_pad,), jnp.float32)

  gather = _make_gather_kernel(n_pad, e_slab)
  scatter = _make_scatter_kernel(n_pad, e_slab)

  bt = 16384
  nb = e_slab // bt
  mlp = pl.pallas_call(
      _mlp_block,
      grid=(nb,),
      in_specs=[
          pl.BlockSpec((1, 1, bt), lambda i: (i, 0, 0)),
          pl.BlockSpec((64, 2), lambda i: (0, 0)),
          pl.BlockSpec((64, 64), lambda i: (0, 0)),
          pl.BlockSpec((2, 64), lambda i: (0, 0)),
      ],
      out_specs=[
          pl.BlockSpec((1, 1, bt), lambda i: (i, 0, 0)),
          pl.BlockSpec((1, 1, bt), lambda i: (i, 0, 0)),
      ],
      out_shape=[
          jax.ShapeDtypeStruct((nb, 1, bt), jnp.float32),
          jax.ShapeDtypeStruct((nb, 1, bt), jnp.float32),
      ],
  )

  parts = []
  cnts = []
  for sl in range(SLABS):
    src_s = lax.slice(src, (sl * e_slab,), ((sl + 1) * e_slab,))
    dst_s = lax.slice(dst, (sl * e_slab,), ((sl + 1) * e_slab,))
    d, cntp = gather(xpacked, src_s, dst_s, zeros_n)
    m0, m1 = mlp(d.reshape(nb, 1, bt), W1.T, W2.T, W3.T)
    (part,) = scatter(m0.reshape(e_slab), m1.reshape(e_slab),
                      dst_s.reshape(e_slab // BATCH, BATCH), zeros_n)
    parts.append(part)
    cnts.append(cntp)

  part_all = jnp.stack(parts)
  cnt_all = jnp.stack(cnts)

  bn = 2048
  outT = pl.pallas_call(
      _combine_block,
      grid=(n_pad // bn,),
      in_specs=[pl.BlockSpec((SLABS, 2, NC, bn), lambda i: (0, 0, 0, i)),
                pl.BlockSpec((SLABS, NC, bn), lambda i: (0, 0, i))],
      out_specs=pl.BlockSpec((2, bn), lambda i: (0, i)),
      out_shape=jax.ShapeDtypeStruct((2, n_pad), jnp.float32),
  )(part_all, cnt_all)

  return outT.T[:n]
